# Initial kernel scaffold; baseline (speedup 1.0000x reference)
import jax
import jax.numpy as jnp
from jax.experimental import pallas as pl


def kernel(x, edge_index_dir, edge_attr, edge_index_und, edge_attr_und, batch, emb, und1, und2, lin1_W, lin1_b, lin2_W, lin2_b, lin3_W, lin3_b):
    raise NotImplementedError("write your pallas kernel here")


# speedup vs baseline: 4.9066x; 4.9066x over previous
import functools

import jax
import jax.numpy as jnp
from jax import lax
from jax.experimental import pallas as pl
from jax.experimental.pallas import tpu as pltpu
from jax.experimental.pallas import tpu_sc as plsc

N = 10000
E = 320000
DH = 128
NG = 16
CH = 80
NW = 32
EW = 16

_HIGH = lax.Precision.HIGHEST


def _sc_mesh():
    return plsc.VectorSubcoreMesh(core_axis_name="c", subcore_axis_name="s")


def _sc_gather(table, idx):
    n = idx.shape[0]
    d = table.shape[1]
    nch = n // CH
    full, rem = divmod(nch, NW)

    @functools.partial(
        pl.kernel,
        mesh=_sc_mesh(),
        out_type=jax.ShapeDtypeStruct((n, d), table.dtype),
        scratch_types=[
            pltpu.VMEM((CH,), jnp.int32),
            pltpu.VMEM((CH, d), table.dtype),
            pltpu.SemaphoreType.DMA,
        ],
    )
    def k(tab_h, idx_h, out_h, idx_v, buf_v, sem):
        wid = lax.axis_index("s") * 2 + lax.axis_index("c")

        def do(c):
            base = c * CH
            pltpu.sync_copy(idx_h.at[pl.ds(base, CH)], idx_v)
            pltpu.async_copy(tab_h.at[idx_v], buf_v, sem).wait()
            pltpu.sync_copy(buf_v, out_h.at[pl.ds(base, CH)])

        if full:
            @pl.loop(0, full)
            def _(i):
                do(wid + i * NW)
        if rem:
            @pl.when(wid < rem)
            def _():
                do(full * NW + wid)

    return k(table, idx)


def _sc_scatter_add(msg, ex, dst, zm):
    nch = E // CH
    per_s = nch // 16

    @functools.partial(
        pl.kernel,
        mesh=_sc_mesh(),
        out_type=[
            jax.ShapeDtypeStruct((N, DH), jnp.float32),
            jax.ShapeDtypeStruct((N, DH), jnp.float32),
        ],
        scratch_types=[
            pltpu.VMEM((CH,), jnp.int32),
            pltpu.VMEM((CH, DH), jnp.float32),
            pltpu.VMEM_SHARED((N, DH), jnp.float32),
            pltpu.SemaphoreType.DMA,
        ],
    )
    def k(msg_h, ex_h, dst_h, zm_h, om_h, oe_h, idx_v, m_v, acc_s, sem):
        cid = lax.axis_index("c")
        sid = lax.axis_index("s")

        @pl.when(sid == 0)
        def _():
            pltpu.sync_copy(zm_h, acc_s)

        plsc.subcore_barrier()

        @pl.loop(0, per_s)
        def _(i):
            base = (sid + i * 16) * CH
            pltpu.sync_copy(dst_h.at[pl.ds(base, CH)], idx_v)

            @pl.when(cid == 0)
            def _():
                pltpu.sync_copy(msg_h.at[pl.ds(base, CH)], m_v)

            @pl.when(cid == 1)
            def _():
                pltpu.sync_copy(ex_h.at[pl.ds(base, CH)], m_v)

            pltpu.sync_copy(m_v, acc_s.at[idx_v], add=True)

        plsc.subcore_barrier()

        @pl.when(sid == 0)
        def _():
            @pl.when(cid == 0)
            def _():
                pltpu.sync_copy(acc_s, om_h)

            @pl.when(cid == 1)
            def _():
                pltpu.sync_copy(acc_s, oe_h)

    return k(msg, ex, dst, zm)


def _qkvs(h, Wcat, bcat8):
    RB = 1000

    def body(h_ref, w_ref, b_ref, q_ref, kv_ref, s_ref):
        acc = jnp.dot(h_ref[...], w_ref[...], precision=_HIGH) + b_ref[0:1, :]
        q_ref[...] = acc[:, :DH]
        kv_ref[...] = acc[:, DH:3 * DH]
        s_ref[...] = acc[:, 3 * DH:]

    return pl.pallas_call(
        body,
        grid=(N // RB,),
        in_specs=[
            pl.BlockSpec((RB, DH), lambda i: (i, 0)),
            pl.BlockSpec((DH, 4 * DH), lambda i: (0, 0)),
            pl.BlockSpec((8, 4 * DH), lambda i: (0, 0)),
        ],
        out_specs=[
            pl.BlockSpec((RB, DH), lambda i: (i, 0)),
            pl.BlockSpec((RB, 2 * DH), lambda i: (i, 0)),
            pl.BlockSpec((RB, DH), lambda i: (i, 0)),
        ],
        out_shape=[
            jax.ShapeDtypeStruct((N, DH), jnp.float32),
            jax.ShapeDtypeStruct((N, 2 * DH), jnp.float32),
            jax.ShapeDtypeStruct((N, DH), jnp.float32),
        ],
    )(h, Wcat, bcat8)


def _edge_math(qd, kvg, attrP, WeP, beP):
    EB = 4000
    inv = 1.0 / (DH ** 0.5)

    def body(qd_ref, kv_ref, at_ref, we_ref, be_ref, msg_ref, ex_ref):
        e = jnp.dot(at_ref[...], we_ref[...], precision=_HIGH) + be_ref[0:1, :]
        kj = kv_ref[:, :DH] + e
        alpha = jnp.sum(qd_ref[...] * kj, axis=1, keepdims=True) * inv
        ex = jnp.exp(alpha)
        msg_ref[...] = (kv_ref[:, DH:] + e) * ex
        ex_ref[...] = ex * jnp.ones((1, DH), jnp.float32)

    return pl.pallas_call(
        body,
        grid=(E // EB,),
        in_specs=[
            pl.BlockSpec((EB, DH), lambda i: (i, 0)),
            pl.BlockSpec((EB, 2 * DH), lambda i: (i, 0)),
            pl.BlockSpec((EB, 8), lambda i: (i, 0)),
            pl.BlockSpec((8, DH), lambda i: (0, 0)),
            pl.BlockSpec((8, DH), lambda i: (0, 0)),
        ],
        out_specs=[
            pl.BlockSpec((EB, DH), lambda i: (i, 0)),
            pl.BlockSpec((EB, DH), lambda i: (i, 0)),
        ],
        out_shape=[
            jax.ShapeDtypeStruct((E, DH), jnp.float32),
            jax.ShapeDtypeStruct((E, DH), jnp.float32),
        ],
    )(qd, kvg, attrP, WeP, beP)


def _combine(accM, accE, s):
    def body(am_ref, ae_ref, s_ref, o_ref):
        denom = ae_ref[:, 0:1]
        o_ref[...] = jnp.maximum(
            am_ref[...] / (denom + 1e-16) + s_ref[...], 0.0)

    return pl.pallas_call(
        body,
        out_shape=jax.ShapeDtypeStruct((N, DH), jnp.float32),
    )(accM, accE, s)


def _readout_mlp(H, batch8, batchb, lin1_W, lin1_b8, lin2_W, lin2_b8,
                 lin3_W8, lin3_b8):
    def body(h_ref, b8_ref, bb_ref, w1_ref, b1_ref, w2_ref, b2_ref,
             w3_ref, b3_ref, o_ref):
        H_ = h_ref[...]
        brow = b8_ref[0:1, :]
        gid = lax.broadcasted_iota(jnp.int32, (NG, 1), 0)
        onehot = (brow == gid).astype(jnp.float32)
        counts = jnp.sum(onehot, axis=1, keepdims=True)
        gsum = jnp.dot(onehot, H_, precision=_HIGH)
        gap = gsum / jnp.maximum(counts, 1.0)
        bb = bb_ref[...]
        neg = jnp.float32(-3.0e38)
        rows = []
        for g in range(NG):
            mg = jnp.where(bb == g, H_, neg)
            rows.append(jnp.max(mg, axis=0, keepdims=True))
        gmp = jnp.concatenate(rows, axis=0)
        gmp = jnp.where(gmp > -1.0e38, gmp, 0.0)
        ro = jnp.concatenate([gap, gmp], axis=1)
        o1 = jnp.maximum(jnp.dot(ro, w1_ref[...], precision=_HIGH)
                         + b1_ref[0:1, :], 0.0)
        o2 = jnp.maximum(jnp.dot(o1, w2_ref[...], precision=_HIGH)
                         + b2_ref[0:1, :], 0.0)
        o3 = jnp.dot(o2, w3_ref[...], precision=_HIGH) + b3_ref[0:1, :]
        o_ref[...] = 1.0 / (1.0 + jnp.exp(-o3))

    return pl.pallas_call(
        body,
        out_shape=jax.ShapeDtypeStruct((NG, 8), jnp.float32),
    )(H, batch8, batchb, lin1_W, lin1_b8, lin2_W, lin2_b8, lin3_W8, lin3_b8)


def _conv_layer(h, src, dst, attrP, p, zm):
    Wcat = jnp.concatenate([p["Wq"], p["Wk"], p["Wv"], p["Ws"]], axis=1)
    bcat = jnp.concatenate([p["bq"], p["bk"], p["bv"], p["bs"]])
    bcat8 = jnp.broadcast_to(bcat[None, :], (8, 4 * DH))
    WeP = jnp.zeros((8, DH), jnp.float32).at[:6].set(p["We"])
    beP = jnp.broadcast_to(p["be"][None, :], (8, DH))

    q, kv, s = _qkvs(h, Wcat, bcat8)
    qd = _sc_gather(q, dst)
    kvg = _sc_gather(kv, src)
    msg, ex = _edge_math(qd, kvg, attrP, WeP, beP)
    accM, accE = _sc_scatter_add(msg, ex, dst, zm)
    return _combine(accM, accE, s)


def kernel(x, edge_index_dir, edge_attr, edge_index_und, edge_attr_und,
           batch, emb, und1, und2, lin1_W, lin1_b, lin2_W, lin2_b,
           lin3_W, lin3_b):
    src = edge_index_und[0].astype(jnp.int32)
    dst = edge_index_und[1].astype(jnp.int32)
    xf = x.reshape(-1).astype(jnp.int32)
    attrP = jnp.concatenate(
        [edge_attr_und, jnp.zeros((E, 2), jnp.float32)], axis=1)
    zm = jnp.zeros((N, DH), jnp.float32)

    emb128 = jnp.concatenate(
        [emb, jnp.zeros((emb.shape[0], DH - emb.shape[1]), jnp.float32)],
        axis=1)
    h0 = _sc_gather(emb128, xf)[:, :32].reshape(N, DH)
    h1 = _conv_layer(h0, src, dst, attrP, und1, zm)
    h2 = _conv_layer(h1, src, dst, attrP, und2, zm)

    batch8 = jnp.broadcast_to(batch.astype(jnp.int32)[None, :], (8, N))
    batchb = jnp.broadcast_to(batch.astype(jnp.int32)[:, None], (N, DH))
    lin1_b8 = jnp.broadcast_to(lin1_b[None, :], (8, 256))
    lin2_b8 = jnp.broadcast_to(lin2_b[None, :], (8, 128))
    lin3_W8 = jnp.broadcast_to(lin3_W, (128, 8))
    lin3_b8 = jnp.broadcast_to(lin3_b[None, :], (8, 8))
    out = _readout_mlp(h2, batch8, batchb, lin1_W, lin1_b8, lin2_W,
                       lin2_b8, lin3_W8, lin3_b8)
    return out[:, 0]


# speedup vs baseline: 6.9251x; 1.4114x over previous
import functools

import jax
import jax.numpy as jnp
from jax import lax
from jax.experimental import pallas as pl
from jax.experimental.pallas import tpu as pltpu
from jax.experimental.pallas import tpu_sc as plsc

N = 10000
E = 320000
DH = 128
NG = 16
CH = 80
NW = 32
EW = 16

_HIGH = lax.Precision.HIGHEST


def _sc_mesh():
    return plsc.VectorSubcoreMesh(core_axis_name="c", subcore_axis_name="s")


def _sc_gather(table, idx):
    n = idx.shape[0]
    d = table.shape[1]
    nch = n // CH
    full, rem = divmod(nch, NW)

    @functools.partial(
        pl.kernel,
        mesh=_sc_mesh(),
        out_type=jax.ShapeDtypeStruct((n, d), table.dtype),
        scratch_types=[
            pltpu.VMEM((CH,), jnp.int32),
            pltpu.VMEM((CH, d), table.dtype),
            pltpu.SemaphoreType.DMA,
        ],
    )
    def k(tab_h, idx_h, out_h, idx_v, buf_v, sem):
        wid = lax.axis_index("s") * 2 + lax.axis_index("c")

        def do(c):
            base = c * CH
            pltpu.sync_copy(idx_h.at[pl.ds(base, CH)], idx_v)
            pltpu.async_copy(tab_h.at[idx_v], buf_v, sem).wait()
            pltpu.sync_copy(buf_v, out_h.at[pl.ds(base, CH)])

        if full:
            @pl.loop(0, full)
            def _(i):
                do(wid + i * NW)
        if rem:
            @pl.when(wid < rem)
            def _():
                do(full * NW + wid)

    return k(table, idx)


GCH = 128
GPW = E // NW
GFULL = GPW // GCH
GTAIL = GPW - GFULL * GCH


def _sc_gather_edges(q, kv, dst, src):
    @functools.partial(
        pl.kernel,
        mesh=_sc_mesh(),
        out_type=[
            jax.ShapeDtypeStruct((E, DH), jnp.float32),
            jax.ShapeDtypeStruct((E, 2 * DH), jnp.float32),
        ],
        scratch_types=[
            pltpu.VMEM((2, GCH), jnp.int32),
            pltpu.VMEM((2, GCH), jnp.int32),
            pltpu.VMEM((2, GCH, DH), jnp.float32),
            pltpu.VMEM((2, GCH, 2 * DH), jnp.float32),
            pltpu.SemaphoreType.DMA,
            pltpu.SemaphoreType.DMA,
            pltpu.SemaphoreType.DMA,
            pltpu.SemaphoreType.DMA,
        ],
    )
    def k(q_h, kv_h, dst_h, src_h, qd_h, kvg_h,
          idxd_v, idxs_v, bq_v, bkv_v, sem0, sem1, sem2, sem3):
        wid = lax.axis_index("s") * 2 + lax.axis_index("c")
        base_w = wid * GPW
        sems = (sem0, sem1, sem2, sem3)

        def loads(base, nrows, b):
            return (pltpu.async_copy(dst_h.at[pl.ds(base, nrows)],
                                     idxd_v.at[b, pl.ds(0, nrows)], sems[b]),
                    pltpu.async_copy(src_h.at[pl.ds(base, nrows)],
                                     idxs_v.at[b, pl.ds(0, nrows)], sems[b]))

        def gathers(nrows, b):
            return (pltpu.async_copy(q_h.at[idxd_v.at[b, pl.ds(0, nrows)]],
                                     bq_v.at[b, pl.ds(0, nrows)], sems[2 + b]),
                    pltpu.async_copy(kv_h.at[idxs_v.at[b, pl.ds(0, nrows)]],
                                     bkv_v.at[b, pl.ds(0, nrows)], sems[2 + b]))

        def writebacks(base, nrows, b):
            return (pltpu.async_copy(bq_v.at[b, pl.ds(0, nrows)],
                                     qd_h.at[pl.ds(base, nrows)], sems[b]),
                    pltpu.async_copy(bkv_v.at[b, pl.ds(0, nrows)],
                                     kvg_h.at[pl.ds(base, nrows)], sems[b]))

        def wait2(hs):
            hs[0].wait()
            hs[1].wait()

        @pl.loop(0, GFULL // 2)
        def _(j):
            bA = base_w + (2 * j) * GCH
            bB = bA + GCH
            lA = loads(bA, GCH, 0)
            lB = loads(bB, GCH, 1)
            wait2(lA)
            gA = gathers(GCH, 0)
            wait2(lB)
            gB = gathers(GCH, 1)
            wait2(gA)
            wA = writebacks(bA, GCH, 0)
            wait2(gB)
            wB = writebacks(bB, GCH, 1)
            wait2(wA)
            wait2(wB)

        def do_single(base, nrows):
            lA = loads(base, nrows, 0)
            wait2(lA)
            gA = gathers(nrows, 0)
            wait2(gA)
            wA = writebacks(base, nrows, 0)
            wait2(wA)

        if GFULL % 2:
            do_single(base_w + (GFULL - 1) * GCH, GCH)
        if GTAIL:
            do_single(base_w + GFULL * GCH, GTAIL)

    return k(q, kv, dst, src)


def _sc_scatter_add(msg, ex, dst, zm):
    SCH = 80
    per_s = E // 16
    nch_s = per_s // SCH
    ROWS15 = 632
    ROWS_LAST = N - 15 * ROWS15

    @functools.partial(
        pl.kernel,
        mesh=_sc_mesh(),
        out_type=[
            jax.ShapeDtypeStruct((N, DH), jnp.float32),
            jax.ShapeDtypeStruct((N, DH), jnp.float32),
        ],
        scratch_types=[
            pltpu.VMEM((2, SCH), jnp.int32),
            pltpu.VMEM((2, SCH, DH), jnp.float32),
            pltpu.VMEM_SHARED((N, DH), jnp.float32),
            pltpu.SemaphoreType.DMA,
            pltpu.SemaphoreType.DMA,
            pltpu.SemaphoreType.DMA,
        ],
    )
    def k(msg_h, ex_h, dst_h, zm_h, om_h, oe_h, idx_v, m_v,
          acc_s, sem0, sem1, sem2):
        cid = lax.axis_index("c")
        sid = lax.axis_index("s")
        sems = (sem0, sem1)
        rbase = sid * ROWS15

        @pl.when(sid < 15)
        def _():
            pltpu.sync_copy(zm_h.at[pl.ds(rbase, ROWS15)],
                            acc_s.at[pl.ds(rbase, ROWS15)])

        @pl.when(sid == 15)
        def _():
            pltpu.sync_copy(zm_h.at[pl.ds(15 * ROWS15, ROWS_LAST)],
                            acc_s.at[pl.ds(15 * ROWS15, ROWS_LAST)])

        plsc.subcore_barrier()

        base0 = sid * per_s

        @pl.loop(0, nch_s // 2)
        def _(j):
            bA = base0 + (2 * j) * SCH
            bB = bA + SCH
            iA = pltpu.async_copy(dst_h.at[pl.ds(bA, SCH)], idx_v.at[0],
                                  sems[0])
            iB = pltpu.async_copy(dst_h.at[pl.ds(bB, SCH)], idx_v.at[1],
                                  sems[1])

            @pl.when(cid == 0)
            def _():
                mA = pltpu.async_copy(msg_h.at[pl.ds(bA, SCH)], m_v.at[0],
                                      sems[0])
                mB = pltpu.async_copy(msg_h.at[pl.ds(bB, SCH)], m_v.at[1],
                                      sems[1])
                iA.wait()
                mA.wait()
                sA = pltpu.async_copy(m_v.at[0], acc_s.at[idx_v.at[0]],
                                      sem2, add=True)
                iB.wait()
                mB.wait()
                sB = pltpu.async_copy(m_v.at[1], acc_s.at[idx_v.at[1]],
                                      sem2, add=True)
                sA.wait()
                sB.wait()

            @pl.when(cid == 1)
            def _():
                mA = pltpu.async_copy(ex_h.at[pl.ds(bA, SCH)], m_v.at[0],
                                      sems[0])
                mB = pltpu.async_copy(ex_h.at[pl.ds(bB, SCH)], m_v.at[1],
                                      sems[1])
                iA.wait()
                mA.wait()
                sA = pltpu.async_copy(m_v.at[0], acc_s.at[idx_v.at[0]],
                                      sem2, add=True)
                iB.wait()
                mB.wait()
                sB = pltpu.async_copy(m_v.at[1], acc_s.at[idx_v.at[1]],
                                      sem2, add=True)
                sA.wait()
                sB.wait()

        plsc.subcore_barrier()

        def drain(o_h):
            @pl.when(sid < 15)
            def _():
                pltpu.sync_copy(acc_s.at[pl.ds(rbase, ROWS15)],
                                o_h.at[pl.ds(rbase, ROWS15)])

            @pl.when(sid == 15)
            def _():
                pltpu.sync_copy(acc_s.at[pl.ds(15 * ROWS15, ROWS_LAST)],
                                o_h.at[pl.ds(15 * ROWS15, ROWS_LAST)])

        @pl.when(cid == 0)
        def _():
            drain(om_h)

        @pl.when(cid == 1)
        def _():
            drain(oe_h)

    return k(msg, ex, dst, zm)


def _qkvs(h, Wcat, bcat8):
    RB = 1000

    def body(h_ref, w_ref, b_ref, q_ref, kv_ref, s_ref):
        acc = jnp.dot(h_ref[...], w_ref[...], precision=_HIGH) + b_ref[0:1, :]
        q_ref[...] = acc[:, :DH]
        kv_ref[...] = acc[:, DH:3 * DH]
        s_ref[...] = acc[:, 3 * DH:]

    return pl.pallas_call(
        body,
        grid=(N // RB,),
        in_specs=[
            pl.BlockSpec((RB, DH), lambda i: (i, 0)),
            pl.BlockSpec((DH, 4 * DH), lambda i: (0, 0)),
            pl.BlockSpec((8, 4 * DH), lambda i: (0, 0)),
        ],
        out_specs=[
            pl.BlockSpec((RB, DH), lambda i: (i, 0)),
            pl.BlockSpec((RB, 2 * DH), lambda i: (i, 0)),
            pl.BlockSpec((RB, DH), lambda i: (i, 0)),
        ],
        out_shape=[
            jax.ShapeDtypeStruct((N, DH), jnp.float32),
            jax.ShapeDtypeStruct((N, 2 * DH), jnp.float32),
            jax.ShapeDtypeStruct((N, DH), jnp.float32),
        ],
    )(h, Wcat, bcat8)


def _edge_math(qd, kvg, attrP, WeP, beP):
    EB = 4000
    inv = 1.0 / (DH ** 0.5)

    def body(qd_ref, kv_ref, at_ref, we_ref, be_ref, msg_ref, ex_ref):
        e = jnp.dot(at_ref[...], we_ref[...], precision=_HIGH) + be_ref[0:1, :]
        kj = kv_ref[:, :DH] + e
        alpha = jnp.sum(qd_ref[...] * kj, axis=1, keepdims=True) * inv
        ex = jnp.exp(alpha)
        msg_ref[...] = (kv_ref[:, DH:] + e) * ex
        ex_ref[...] = ex * jnp.ones((1, DH), jnp.float32)

    return pl.pallas_call(
        body,
        grid=(E // EB,),
        in_specs=[
            pl.BlockSpec((EB, DH), lambda i: (i, 0)),
            pl.BlockSpec((EB, 2 * DH), lambda i: (i, 0)),
            pl.BlockSpec((EB, 8), lambda i: (i, 0)),
            pl.BlockSpec((8, DH), lambda i: (0, 0)),
            pl.BlockSpec((8, DH), lambda i: (0, 0)),
        ],
        out_specs=[
            pl.BlockSpec((EB, DH), lambda i: (i, 0)),
            pl.BlockSpec((EB, DH), lambda i: (i, 0)),
        ],
        out_shape=[
            jax.ShapeDtypeStruct((E, DH), jnp.float32),
            jax.ShapeDtypeStruct((E, DH), jnp.float32),
        ],
    )(qd, kvg, attrP, WeP, beP)


def _combine(accM, accE, s):
    def body(am_ref, ae_ref, s_ref, o_ref):
        denom = ae_ref[:, 0:1]
        o_ref[...] = jnp.maximum(
            am_ref[...] / (denom + 1e-16) + s_ref[...], 0.0)

    return pl.pallas_call(
        body,
        out_shape=jax.ShapeDtypeStruct((N, DH), jnp.float32),
    )(accM, accE, s)


def _readout_mlp(H, batch8, batchb, lin1_W, lin1_b8, lin2_W, lin2_b8,
                 lin3_W8, lin3_b8):
    def body(h_ref, b8_ref, bb_ref, w1_ref, b1_ref, w2_ref, b2_ref,
             w3_ref, b3_ref, o_ref):
        H_ = h_ref[...]
        brow = b8_ref[0:1, :]
        gid = lax.broadcasted_iota(jnp.int32, (NG, 1), 0)
        onehot = (brow == gid).astype(jnp.float32)
        counts = jnp.sum(onehot, axis=1, keepdims=True)
        gsum = jnp.dot(onehot, H_, precision=_HIGH)
        gap = gsum / jnp.maximum(counts, 1.0)
        bb = bb_ref[...]
        neg = jnp.float32(-3.0e38)
        rows = []
        for g in range(NG):
            mg = jnp.where(bb == g, H_, neg)
            rows.append(jnp.max(mg, axis=0, keepdims=True))
        gmp = jnp.concatenate(rows, axis=0)
        gmp = jnp.where(gmp > -1.0e38, gmp, 0.0)
        ro = jnp.concatenate([gap, gmp], axis=1)
        o1 = jnp.maximum(jnp.dot(ro, w1_ref[...], precision=_HIGH)
                         + b1_ref[0:1, :], 0.0)
        o2 = jnp.maximum(jnp.dot(o1, w2_ref[...], precision=_HIGH)
                         + b2_ref[0:1, :], 0.0)
        o3 = jnp.dot(o2, w3_ref[...], precision=_HIGH) + b3_ref[0:1, :]
        o_ref[...] = 1.0 / (1.0 + jnp.exp(-o3))

    return pl.pallas_call(
        body,
        out_shape=jax.ShapeDtypeStruct((NG, 8), jnp.float32),
    )(H, batch8, batchb, lin1_W, lin1_b8, lin2_W, lin2_b8, lin3_W8, lin3_b8)


def _conv_layer(h, src, dst, attrP, p, zm):
    Wcat = jnp.concatenate([p["Wq"], p["Wk"], p["Wv"], p["Ws"]], axis=1)
    bcat = jnp.concatenate([p["bq"], p["bk"], p["bv"], p["bs"]])
    bcat8 = jnp.broadcast_to(bcat[None, :], (8, 4 * DH))
    WeP = jnp.zeros((8, DH), jnp.float32).at[:6].set(p["We"])
    beP = jnp.broadcast_to(p["be"][None, :], (8, DH))

    q, kv, s = _qkvs(h, Wcat, bcat8)
    qd, kvg = _sc_gather_edges(q, kv, dst, src)
    msg, ex = _edge_math(qd, kvg, attrP, WeP, beP)
    accM, accE = _sc_scatter_add(msg, ex, dst, zm)
    return _combine(accM, accE, s)


def kernel(x, edge_index_dir, edge_attr, edge_index_und, edge_attr_und,
           batch, emb, und1, und2, lin1_W, lin1_b, lin2_W, lin2_b,
           lin3_W, lin3_b):
    src = edge_index_und[0].astype(jnp.int32)
    dst = edge_index_und[1].astype(jnp.int32)
    xf = x.reshape(-1).astype(jnp.int32)
    attrP = jnp.concatenate(
        [edge_attr_und, jnp.zeros((E, 2), jnp.float32)], axis=1)
    zm = jnp.zeros((N, DH), jnp.float32)

    emb128 = jnp.concatenate(
        [emb, jnp.zeros((emb.shape[0], DH - emb.shape[1]), jnp.float32)],
        axis=1)
    h0 = _sc_gather(emb128, xf)[:, :32].reshape(N, DH)
    h1 = _conv_layer(h0, src, dst, attrP, und1, zm)
    h2 = _conv_layer(h1, src, dst, attrP, und2, zm)

    batch8 = jnp.broadcast_to(batch.astype(jnp.int32)[None, :], (8, N))
    batchb = jnp.broadcast_to(batch.astype(jnp.int32)[:, None], (N, DH))
    lin1_b8 = jnp.broadcast_to(lin1_b[None, :], (8, 256))
    lin2_b8 = jnp.broadcast_to(lin2_b[None, :], (8, 128))
    lin3_W8 = jnp.broadcast_to(lin3_W, (128, 8))
    lin3_b8 = jnp.broadcast_to(lin3_b[None, :], (8, 8))
    out = _readout_mlp(h2, batch8, batchb, lin1_W, lin1_b8, lin2_W,
                       lin2_b8, lin3_W8, lin3_b8)
    return out[:, 0]


# speedup vs baseline: 7.5538x; 1.0908x over previous
import functools

import jax
import jax.numpy as jnp
from jax import lax
from jax.experimental import pallas as pl
from jax.experimental.pallas import tpu as pltpu
from jax.experimental.pallas import tpu_sc as plsc

N = 10000
E = 320000
DH = 128
NG = 16
CH = 80
NW = 32
EW = 16

_HIGH = lax.Precision.HIGHEST


def _sc_mesh():
    return plsc.VectorSubcoreMesh(core_axis_name="c", subcore_axis_name="s")


def _sc_gather(table, idx):
    n = idx.shape[0]
    d = table.shape[1]
    nch = n // CH
    full, rem = divmod(nch, NW)

    @functools.partial(
        pl.kernel,
        mesh=_sc_mesh(),
        out_type=jax.ShapeDtypeStruct((n, d), table.dtype),
        scratch_types=[
            pltpu.VMEM((CH,), jnp.int32),
            pltpu.VMEM((CH, d), table.dtype),
            pltpu.SemaphoreType.DMA,
        ],
    )
    def k(tab_h, idx_h, out_h, idx_v, buf_v, sem):
        wid = lax.axis_index("s") * 2 + lax.axis_index("c")

        def do(c):
            base = c * CH
            pltpu.sync_copy(idx_h.at[pl.ds(base, CH)], idx_v)
            pltpu.async_copy(tab_h.at[idx_v], buf_v, sem).wait()
            pltpu.sync_copy(buf_v, out_h.at[pl.ds(base, CH)])

        if full:
            @pl.loop(0, full)
            def _(i):
                do(wid + i * NW)
        if rem:
            @pl.when(wid < rem)
            def _():
                do(full * NW + wid)

    return k(table, idx)


GCH = 128


def _sc_gather_edges(q, kv, dst, src):
    ne = dst.shape[0]
    GPW = ne // NW
    GFULL = GPW // GCH
    GTAIL = GPW - GFULL * GCH

    @functools.partial(
        pl.kernel,
        mesh=_sc_mesh(),
        out_type=[
            jax.ShapeDtypeStruct((ne, DH), jnp.float32),
            jax.ShapeDtypeStruct((ne, 2 * DH), jnp.float32),
        ],
        scratch_types=[
            pltpu.VMEM((2, GCH), jnp.int32),
            pltpu.VMEM((2, GCH), jnp.int32),
            pltpu.VMEM((2, GCH, DH), jnp.float32),
            pltpu.VMEM((2, GCH, 2 * DH), jnp.float32),
            pltpu.SemaphoreType.DMA,
            pltpu.SemaphoreType.DMA,
            pltpu.SemaphoreType.DMA,
            pltpu.SemaphoreType.DMA,
        ],
    )
    def k(q_h, kv_h, dst_h, src_h, qd_h, kvg_h,
          idxd_v, idxs_v, bq_v, bkv_v, sem0, sem1, sem2, sem3):
        wid = lax.axis_index("s") * 2 + lax.axis_index("c")
        base_w = wid * GPW
        sems = (sem0, sem1, sem2, sem3)

        def loads(base, nrows, b):
            return (pltpu.async_copy(dst_h.at[pl.ds(base, nrows)],
                                     idxd_v.at[b, pl.ds(0, nrows)], sems[b]),
                    pltpu.async_copy(src_h.at[pl.ds(base, nrows)],
                                     idxs_v.at[b, pl.ds(0, nrows)], sems[b]))

        def gathers(nrows, b):
            return (pltpu.async_copy(q_h.at[idxd_v.at[b, pl.ds(0, nrows)]],
                                     bq_v.at[b, pl.ds(0, nrows)], sems[2 + b]),
                    pltpu.async_copy(kv_h.at[idxs_v.at[b, pl.ds(0, nrows)]],
                                     bkv_v.at[b, pl.ds(0, nrows)], sems[2 + b]))

        def writebacks(base, nrows, b):
            return (pltpu.async_copy(bq_v.at[b, pl.ds(0, nrows)],
                                     qd_h.at[pl.ds(base, nrows)], sems[b]),
                    pltpu.async_copy(bkv_v.at[b, pl.ds(0, nrows)],
                                     kvg_h.at[pl.ds(base, nrows)], sems[b]))

        def wait2(hs):
            hs[0].wait()
            hs[1].wait()

        @pl.loop(0, GFULL // 2)
        def _(j):
            bA = base_w + (2 * j) * GCH
            bB = bA + GCH
            lA = loads(bA, GCH, 0)
            lB = loads(bB, GCH, 1)
            wait2(lA)
            gA = gathers(GCH, 0)
            wait2(lB)
            gB = gathers(GCH, 1)
            wait2(gA)
            wA = writebacks(bA, GCH, 0)
            wait2(gB)
            wB = writebacks(bB, GCH, 1)
            wait2(wA)
            wait2(wB)

        def do_single(base, nrows):
            lA = loads(base, nrows, 0)
            wait2(lA)
            gA = gathers(nrows, 0)
            wait2(gA)
            wA = writebacks(base, nrows, 0)
            wait2(wA)

        if GFULL % 2:
            do_single(base_w + (GFULL - 1) * GCH, GCH)
        if GTAIL:
            do_single(base_w + GFULL * GCH, GTAIL)

    return k(q, kv, dst, src)


def _sc_scatter_add(msg, ex, dst, zm):
    ne = dst.shape[0]
    SCH = 80
    per_s = ne // 16
    nch_s = per_s // SCH
    ROWS15 = 632
    ROWS_LAST = N - 15 * ROWS15

    @functools.partial(
        pl.kernel,
        mesh=_sc_mesh(),
        out_type=[
            jax.ShapeDtypeStruct((N, DH), jnp.float32),
            jax.ShapeDtypeStruct((N, DH), jnp.float32),
        ],
        scratch_types=[
            pltpu.VMEM((2, SCH), jnp.int32),
            pltpu.VMEM((2, SCH, DH), jnp.float32),
            pltpu.VMEM_SHARED((N, DH), jnp.float32),
            pltpu.SemaphoreType.DMA,
            pltpu.SemaphoreType.DMA,
            pltpu.SemaphoreType.DMA,
        ],
    )
    def k(msg_h, ex_h, dst_h, zm_h, om_h, oe_h, idx_v, m_v,
          acc_s, sem0, sem1, sem2):
        cid = lax.axis_index("c")
        sid = lax.axis_index("s")
        sems = (sem0, sem1)
        rbase = sid * ROWS15

        @pl.when(sid < 15)
        def _():
            pltpu.sync_copy(zm_h.at[pl.ds(rbase, ROWS15)],
                            acc_s.at[pl.ds(rbase, ROWS15)])

        @pl.when(sid == 15)
        def _():
            pltpu.sync_copy(zm_h.at[pl.ds(15 * ROWS15, ROWS_LAST)],
                            acc_s.at[pl.ds(15 * ROWS15, ROWS_LAST)])

        plsc.subcore_barrier()

        base0 = sid * per_s

        @pl.loop(0, nch_s // 2)
        def _(j):
            bA = base0 + (2 * j) * SCH
            bB = bA + SCH
            iA = pltpu.async_copy(dst_h.at[pl.ds(bA, SCH)], idx_v.at[0],
                                  sems[0])
            iB = pltpu.async_copy(dst_h.at[pl.ds(bB, SCH)], idx_v.at[1],
                                  sems[1])

            @pl.when(cid == 0)
            def _():
                mA = pltpu.async_copy(msg_h.at[pl.ds(bA, SCH)], m_v.at[0],
                                      sems[0])
                mB = pltpu.async_copy(msg_h.at[pl.ds(bB, SCH)], m_v.at[1],
                                      sems[1])
                iA.wait()
                mA.wait()
                sA = pltpu.async_copy(m_v.at[0], acc_s.at[idx_v.at[0]],
                                      sem2, add=True)
                iB.wait()
                mB.wait()
                sB = pltpu.async_copy(m_v.at[1], acc_s.at[idx_v.at[1]],
                                      sem2, add=True)
                sA.wait()
                sB.wait()

            @pl.when(cid == 1)
            def _():
                mA = pltpu.async_copy(ex_h.at[pl.ds(bA, SCH)], m_v.at[0],
                                      sems[0])
                mB = pltpu.async_copy(ex_h.at[pl.ds(bB, SCH)], m_v.at[1],
                                      sems[1])
                iA.wait()
                mA.wait()
                sA = pltpu.async_copy(m_v.at[0], acc_s.at[idx_v.at[0]],
                                      sem2, add=True)
                iB.wait()
                mB.wait()
                sB = pltpu.async_copy(m_v.at[1], acc_s.at[idx_v.at[1]],
                                      sem2, add=True)
                sA.wait()
                sB.wait()

        if nch_s % 2:
            bA = base0 + (nch_s - 1) * SCH
            iA = pltpu.async_copy(dst_h.at[pl.ds(bA, SCH)], idx_v.at[0],
                                  sems[0])

            @pl.when(cid == 0)
            def _():
                mA = pltpu.async_copy(msg_h.at[pl.ds(bA, SCH)], m_v.at[0],
                                      sems[0])
                iA.wait()
                mA.wait()
                pltpu.sync_copy(m_v.at[0], acc_s.at[idx_v.at[0]], add=True)

            @pl.when(cid == 1)
            def _():
                mA = pltpu.async_copy(ex_h.at[pl.ds(bA, SCH)], m_v.at[0],
                                      sems[0])
                iA.wait()
                mA.wait()
                pltpu.sync_copy(m_v.at[0], acc_s.at[idx_v.at[0]], add=True)

        plsc.subcore_barrier()

        def drain(o_h):
            @pl.when(sid < 15)
            def _():
                pltpu.sync_copy(acc_s.at[pl.ds(rbase, ROWS15)],
                                o_h.at[pl.ds(rbase, ROWS15)])

            @pl.when(sid == 15)
            def _():
                pltpu.sync_copy(acc_s.at[pl.ds(15 * ROWS15, ROWS_LAST)],
                                o_h.at[pl.ds(15 * ROWS15, ROWS_LAST)])

        @pl.when(cid == 0)
        def _():
            drain(om_h)

        @pl.when(cid == 1)
        def _():
            drain(oe_h)

    return k(msg, ex, dst, zm)


def _qkvs(h, Wcat, bcat8):
    RB = 1000

    def body(h_ref, w_ref, b_ref, q_ref, kv_ref, s_ref):
        acc = jnp.dot(h_ref[...], w_ref[...], precision=_HIGH) + b_ref[0:1, :]
        q_ref[...] = acc[:, :DH]
        kv_ref[...] = acc[:, DH:3 * DH]
        s_ref[...] = acc[:, 3 * DH:]

    return pl.pallas_call(
        body,
        grid=(N // RB,),
        in_specs=[
            pl.BlockSpec((RB, DH), lambda i: (i, 0)),
            pl.BlockSpec((DH, 4 * DH), lambda i: (0, 0)),
            pl.BlockSpec((8, 4 * DH), lambda i: (0, 0)),
        ],
        out_specs=[
            pl.BlockSpec((RB, DH), lambda i: (i, 0)),
            pl.BlockSpec((RB, 2 * DH), lambda i: (i, 0)),
            pl.BlockSpec((RB, DH), lambda i: (i, 0)),
        ],
        out_shape=[
            jax.ShapeDtypeStruct((N, DH), jnp.float32),
            jax.ShapeDtypeStruct((N, 2 * DH), jnp.float32),
            jax.ShapeDtypeStruct((N, DH), jnp.float32),
        ],
    )(h, Wcat, bcat8)


def _edge_math(qd, kvg, attrP, WeP, beP):
    ne = qd.shape[0]
    EB = 4000
    inv = 1.0 / (DH ** 0.5)

    def body(qd_ref, kv_ref, at_ref, we_ref, be_ref, msg_ref, ex_ref):
        e = jnp.dot(at_ref[...], we_ref[...], precision=_HIGH) + be_ref[0:1, :]
        kj = kv_ref[:, :DH] + e
        alpha = jnp.sum(qd_ref[...] * kj, axis=1, keepdims=True) * inv
        ex = jnp.exp(alpha)
        msg_ref[...] = (kv_ref[:, DH:] + e) * ex
        ex_ref[...] = ex * jnp.ones((1, DH), jnp.float32)

    return pl.pallas_call(
        body,
        grid=(ne // EB,),
        in_specs=[
            pl.BlockSpec((EB, DH), lambda i: (i, 0)),
            pl.BlockSpec((EB, 2 * DH), lambda i: (i, 0)),
            pl.BlockSpec((EB, 8), lambda i: (i, 0)),
            pl.BlockSpec((8, DH), lambda i: (0, 0)),
            pl.BlockSpec((8, DH), lambda i: (0, 0)),
        ],
        out_specs=[
            pl.BlockSpec((EB, DH), lambda i: (i, 0)),
            pl.BlockSpec((EB, DH), lambda i: (i, 0)),
        ],
        out_shape=[
            jax.ShapeDtypeStruct((ne, DH), jnp.float32),
            jax.ShapeDtypeStruct((ne, DH), jnp.float32),
        ],
    )(qd, kvg, attrP, WeP, beP)


def _combine(omA, omB, oeA, oeB, s):
    def body(a_ref, b_ref, c_ref, d_ref, s_ref, o_ref):
        denom = c_ref[:, 0:1] + d_ref[:, 0:1]
        o_ref[...] = jnp.maximum(
            (a_ref[...] + b_ref[...]) / (denom + 1e-16) + s_ref[...], 0.0)

    return pl.pallas_call(
        body,
        out_shape=jax.ShapeDtypeStruct((N, DH), jnp.float32),
    )(omA, omB, oeA, oeB, s)


def _readout_mlp(H, batch8, batchb, lin1_W, lin1_b8, lin2_W, lin2_b8,
                 lin3_W8, lin3_b8):
    def body(h_ref, b8_ref, bb_ref, w1_ref, b1_ref, w2_ref, b2_ref,
             w3_ref, b3_ref, o_ref):
        H_ = h_ref[...]
        brow = b8_ref[0:1, :]
        gid = lax.broadcasted_iota(jnp.int32, (NG, 1), 0)
        onehot = (brow == gid).astype(jnp.float32)
        counts = jnp.sum(onehot, axis=1, keepdims=True)
        gsum = jnp.dot(onehot, H_, precision=_HIGH)
        gap = gsum / jnp.maximum(counts, 1.0)
        bb = bb_ref[...]
        neg = jnp.float32(-3.0e38)
        rows = []
        for g in range(NG):
            mg = jnp.where(bb == g, H_, neg)
            rows.append(jnp.max(mg, axis=0, keepdims=True))
        gmp = jnp.concatenate(rows, axis=0)
        gmp = jnp.where(gmp > -1.0e38, gmp, 0.0)
        ro = jnp.concatenate([gap, gmp], axis=1)
        o1 = jnp.maximum(jnp.dot(ro, w1_ref[...], precision=_HIGH)
                         + b1_ref[0:1, :], 0.0)
        o2 = jnp.maximum(jnp.dot(o1, w2_ref[...], precision=_HIGH)
                         + b2_ref[0:1, :], 0.0)
        o3 = jnp.dot(o2, w3_ref[...], precision=_HIGH) + b3_ref[0:1, :]
        o_ref[...] = 1.0 / (1.0 + jnp.exp(-o3))

    return pl.pallas_call(
        body,
        out_shape=jax.ShapeDtypeStruct((NG, 8), jnp.float32),
    )(H, batch8, batchb, lin1_W, lin1_b8, lin2_W, lin2_b8, lin3_W8, lin3_b8)


def _conv_layer(h, srcs, dsts, attrs, p, zm):
    Wcat = jnp.concatenate([p["Wq"], p["Wk"], p["Wv"], p["Ws"]], axis=1)
    bcat = jnp.concatenate([p["bq"], p["bk"], p["bv"], p["bs"]])
    bcat8 = jnp.broadcast_to(bcat[None, :], (8, 4 * DH))
    WeP = jnp.zeros((8, DH), jnp.float32).at[:6].set(p["We"])
    beP = jnp.broadcast_to(p["be"][None, :], (8, DH))

    q, kv, s = _qkvs(h, Wcat, bcat8)
    qd1, kvg1 = _sc_gather_edges(q, kv, dsts[0], srcs[0])
    msg1, ex1 = _edge_math(qd1, kvg1, attrs[0], WeP, beP)
    qd2, kvg2 = _sc_gather_edges(q, kv, dsts[1], srcs[1])
    omA, oeA = _sc_scatter_add(msg1, ex1, dsts[0], zm)
    msg2, ex2 = _edge_math(qd2, kvg2, attrs[1], WeP, beP)
    omB, oeB = _sc_scatter_add(msg2, ex2, dsts[1], zm)
    return _combine(omA, omB, oeA, oeB, s)


def kernel(x, edge_index_dir, edge_attr, edge_index_und, edge_attr_und,
           batch, emb, und1, und2, lin1_W, lin1_b, lin2_W, lin2_b,
           lin3_W, lin3_b):
    src = edge_index_und[0].astype(jnp.int32)
    dst = edge_index_und[1].astype(jnp.int32)
    xf = x.reshape(-1).astype(jnp.int32)
    attrP = jnp.concatenate(
        [edge_attr_und, jnp.zeros((E, 2), jnp.float32)], axis=1)
    E2 = E // 2
    srcs = (src[:E2], src[E2:])
    dsts = (dst[:E2], dst[E2:])
    attrs = (attrP[:E2], attrP[E2:])
    zm = jnp.zeros((N, DH), jnp.float32)

    emb128 = jnp.concatenate(
        [emb, jnp.zeros((emb.shape[0], DH - emb.shape[1]), jnp.float32)],
        axis=1)
    h0 = _sc_gather(emb128, xf)[:, :32].reshape(N, DH)
    h1 = _conv_layer(h0, srcs, dsts, attrs, und1, zm)
    h2 = _conv_layer(h1, srcs, dsts, attrs, und2, zm)

    batch8 = jnp.broadcast_to(batch.astype(jnp.int32)[None, :], (8, N))
    batchb = jnp.broadcast_to(batch.astype(jnp.int32)[:, None], (N, DH))
    lin1_b8 = jnp.broadcast_to(lin1_b[None, :], (8, 256))
    lin2_b8 = jnp.broadcast_to(lin2_b[None, :], (8, 128))
    lin3_W8 = jnp.broadcast_to(lin3_W, (128, 8))
    lin3_b8 = jnp.broadcast_to(lin3_b[None, :], (8, 8))
    out = _readout_mlp(h2, batch8, batchb, lin1_W, lin1_b8, lin2_W,
                       lin2_b8, lin3_W8, lin3_b8)
    return out[:, 0]


# speedup vs baseline: 7.6050x; 1.0068x over previous
import functools

import jax
import jax.numpy as jnp
from jax import lax
from jax.experimental import pallas as pl
from jax.experimental.pallas import tpu as pltpu
from jax.experimental.pallas import tpu_sc as plsc

N = 10000
E = 320000
DH = 128
NG = 16
CH = 80
NW = 32
EW = 16

_HIGH = lax.Precision.HIGHEST


def _sc_mesh():
    return plsc.VectorSubcoreMesh(core_axis_name="c", subcore_axis_name="s")


def _sc_gather(table, idx):
    n = idx.shape[0]
    d = table.shape[1]
    nch = n // CH
    full, rem = divmod(nch, NW)

    @functools.partial(
        pl.kernel,
        mesh=_sc_mesh(),
        out_type=jax.ShapeDtypeStruct((n, d), table.dtype),
        scratch_types=[
            pltpu.VMEM((CH,), jnp.int32),
            pltpu.VMEM((CH, d), table.dtype),
            pltpu.SemaphoreType.DMA,
        ],
    )
    def k(tab_h, idx_h, out_h, idx_v, buf_v, sem):
        wid = lax.axis_index("s") * 2 + lax.axis_index("c")

        def do(c):
            base = c * CH
            pltpu.sync_copy(idx_h.at[pl.ds(base, CH)], idx_v)
            pltpu.async_copy(tab_h.at[idx_v], buf_v, sem).wait()
            pltpu.sync_copy(buf_v, out_h.at[pl.ds(base, CH)])

        if full:
            @pl.loop(0, full)
            def _(i):
                do(wid + i * NW)
        if rem:
            @pl.when(wid < rem)
            def _():
                do(full * NW + wid)

    return k(table, idx)


GCH = 128


def _sc_gather_edges(q, kv, dst, src):
    ne = dst.shape[0]
    GPW = ne // NW
    GFULL = GPW // GCH
    GTAIL = GPW - GFULL * GCH

    @functools.partial(
        pl.kernel,
        mesh=_sc_mesh(),
        out_type=[
            jax.ShapeDtypeStruct((ne, DH), jnp.float32),
            jax.ShapeDtypeStruct((ne, 2 * DH), jnp.float32),
        ],
        scratch_types=[
            pltpu.VMEM((2, GCH), jnp.int32),
            pltpu.VMEM((2, GCH), jnp.int32),
            pltpu.VMEM((2, GCH, DH), jnp.float32),
            pltpu.VMEM((2, GCH, 2 * DH), jnp.float32),
            pltpu.SemaphoreType.DMA,
            pltpu.SemaphoreType.DMA,
            pltpu.SemaphoreType.DMA,
            pltpu.SemaphoreType.DMA,
            pltpu.SemaphoreType.DMA,
            pltpu.SemaphoreType.DMA,
        ],
    )
    def k(q_h, kv_h, dst_h, src_h, qd_h, kvg_h,
          idxd_v, idxs_v, bq_v, bkv_v, sl0, sl1, sg0, sg1, sw0, sw1):
        wid = lax.axis_index("s") * 2 + lax.axis_index("c")
        base_w = wid * GPW
        sl = (sl0, sl1)
        sg = (sg0, sg1)
        sw = (sw0, sw1)

        def loads(base, nrows, b):
            return (pltpu.async_copy(dst_h.at[pl.ds(base, nrows)],
                                     idxd_v.at[b, pl.ds(0, nrows)], sl[b]),
                    pltpu.async_copy(src_h.at[pl.ds(base, nrows)],
                                     idxs_v.at[b, pl.ds(0, nrows)], sl[b]))

        def gathers(nrows, b):
            return (pltpu.async_copy(q_h.at[idxd_v.at[b, pl.ds(0, nrows)]],
                                     bq_v.at[b, pl.ds(0, nrows)], sg[b]),
                    pltpu.async_copy(kv_h.at[idxs_v.at[b, pl.ds(0, nrows)]],
                                     bkv_v.at[b, pl.ds(0, nrows)], sg[b]))

        def writebacks(base, nrows, b):
            return (pltpu.async_copy(bq_v.at[b, pl.ds(0, nrows)],
                                     qd_h.at[pl.ds(base, nrows)], sw[b]),
                    pltpu.async_copy(bkv_v.at[b, pl.ds(0, nrows)],
                                     kvg_h.at[pl.ds(base, nrows)], sw[b]))

        def drain_wb(b):
            pltpu.make_async_copy(bq_v.at[b, pl.ds(0, GCH)],
                                  qd_h.at[pl.ds(0, GCH)], sw[b]).wait()
            pltpu.make_async_copy(bkv_v.at[b, pl.ds(0, GCH)],
                                  kvg_h.at[pl.ds(0, GCH)], sw[b]).wait()

        def wait2(hs):
            hs[0].wait()
            hs[1].wait()

        def pair(bA, first):
            bB = bA + GCH
            lA = loads(bA, GCH, 0)
            lB = loads(bB, GCH, 1)
            wait2(lA)
            if not first:
                drain_wb(0)
            gA = gathers(GCH, 0)
            wait2(lB)
            if not first:
                drain_wb(1)
            gB = gathers(GCH, 1)
            wait2(gA)
            writebacks(bA, GCH, 0)
            wait2(gB)
            writebacks(bB, GCH, 1)

        NP = GFULL // 2
        pair(base_w, True)

        @pl.loop(1, NP)
        def _(j):
            pair(base_w + (2 * j) * GCH, False)

        def do_single(base, nrows, drain):
            lA = loads(base, nrows, 0)
            wait2(lA)
            if drain:
                drain_wb(0)
            gA = gathers(nrows, 0)
            wait2(gA)
            wA = writebacks(base, nrows, 0)
            wait2(wA)

        if GFULL % 2:
            do_single(base_w + (GFULL - 1) * GCH, GCH, True)
            if GTAIL:
                do_single(base_w + GFULL * GCH, GTAIL, False)
            drain_wb(1)
        else:
            if GTAIL:
                do_single(base_w + GFULL * GCH, GTAIL, True)
                drain_wb(1)
            else:
                drain_wb(0)
                drain_wb(1)

    return k(q, kv, dst, src)


def _sc_scatter_add(msg, ex, dst, zm):
    ne = dst.shape[0]
    SCH = 80
    per_s = ne // 16
    nch_s = per_s // SCH
    ROWS15 = 632
    ROWS_LAST = N - 15 * ROWS15

    @functools.partial(
        pl.kernel,
        mesh=_sc_mesh(),
        out_type=[
            jax.ShapeDtypeStruct((N, DH), jnp.float32),
            jax.ShapeDtypeStruct((N, DH), jnp.float32),
        ],
        scratch_types=[
            pltpu.VMEM((2, SCH), jnp.int32),
            pltpu.VMEM((2, SCH, DH), jnp.float32),
            pltpu.VMEM_SHARED((N, DH), jnp.float32),
            pltpu.SemaphoreType.DMA,
            pltpu.SemaphoreType.DMA,
            pltpu.SemaphoreType.DMA,
        ],
    )
    def k(msg_h, ex_h, dst_h, zm_h, om_h, oe_h, idx_v, m_v,
          acc_s, sem0, sem1, sem2):
        cid = lax.axis_index("c")
        sid = lax.axis_index("s")
        sems = (sem0, sem1)
        rbase = sid * ROWS15

        @pl.when(sid < 15)
        def _():
            pltpu.sync_copy(zm_h.at[pl.ds(rbase, ROWS15)],
                            acc_s.at[pl.ds(rbase, ROWS15)])

        @pl.when(sid == 15)
        def _():
            pltpu.sync_copy(zm_h.at[pl.ds(15 * ROWS15, ROWS_LAST)],
                            acc_s.at[pl.ds(15 * ROWS15, ROWS_LAST)])

        plsc.subcore_barrier()

        base0 = sid * per_s

        @pl.loop(0, nch_s // 2)
        def _(j):
            bA = base0 + (2 * j) * SCH
            bB = bA + SCH
            iA = pltpu.async_copy(dst_h.at[pl.ds(bA, SCH)], idx_v.at[0],
                                  sems[0])
            iB = pltpu.async_copy(dst_h.at[pl.ds(bB, SCH)], idx_v.at[1],
                                  sems[1])

            @pl.when(cid == 0)
            def _():
                mA = pltpu.async_copy(msg_h.at[pl.ds(bA, SCH)], m_v.at[0],
                                      sems[0])
                mB = pltpu.async_copy(msg_h.at[pl.ds(bB, SCH)], m_v.at[1],
                                      sems[1])
                iA.wait()
                mA.wait()
                sA = pltpu.async_copy(m_v.at[0], acc_s.at[idx_v.at[0]],
                                      sem2, add=True)
                iB.wait()
                mB.wait()
                sB = pltpu.async_copy(m_v.at[1], acc_s.at[idx_v.at[1]],
                                      sem2, add=True)
                sA.wait()
                sB.wait()

            @pl.when(cid == 1)
            def _():
                mA = pltpu.async_copy(ex_h.at[pl.ds(bA, SCH)], m_v.at[0],
                                      sems[0])
                mB = pltpu.async_copy(ex_h.at[pl.ds(bB, SCH)], m_v.at[1],
                                      sems[1])
                iA.wait()
                mA.wait()
                sA = pltpu.async_copy(m_v.at[0], acc_s.at[idx_v.at[0]],
                                      sem2, add=True)
                iB.wait()
                mB.wait()
                sB = pltpu.async_copy(m_v.at[1], acc_s.at[idx_v.at[1]],
                                      sem2, add=True)
                sA.wait()
                sB.wait()

        if nch_s % 2:
            bA = base0 + (nch_s - 1) * SCH
            iA = pltpu.async_copy(dst_h.at[pl.ds(bA, SCH)], idx_v.at[0],
                                  sems[0])

            @pl.when(cid == 0)
            def _():
                mA = pltpu.async_copy(msg_h.at[pl.ds(bA, SCH)], m_v.at[0],
                                      sems[0])
                iA.wait()
                mA.wait()
                pltpu.sync_copy(m_v.at[0], acc_s.at[idx_v.at[0]], add=True)

            @pl.when(cid == 1)
            def _():
                mA = pltpu.async_copy(ex_h.at[pl.ds(bA, SCH)], m_v.at[0],
                                      sems[0])
                iA.wait()
                mA.wait()
                pltpu.sync_copy(m_v.at[0], acc_s.at[idx_v.at[0]], add=True)

        plsc.subcore_barrier()

        def drain(o_h):
            @pl.when(sid < 15)
            def _():
                pltpu.sync_copy(acc_s.at[pl.ds(rbase, ROWS15)],
                                o_h.at[pl.ds(rbase, ROWS15)])

            @pl.when(sid == 15)
            def _():
                pltpu.sync_copy(acc_s.at[pl.ds(15 * ROWS15, ROWS_LAST)],
                                o_h.at[pl.ds(15 * ROWS15, ROWS_LAST)])

        @pl.when(cid == 0)
        def _():
            drain(om_h)

        @pl.when(cid == 1)
        def _():
            drain(oe_h)

    return k(msg, ex, dst, zm)


def _qkvs(h, Wcat, bcat8):
    RB = 1000

    def body(h_ref, w_ref, b_ref, q_ref, kv_ref, s_ref):
        acc = jnp.dot(h_ref[...], w_ref[...], precision=_HIGH) + b_ref[0:1, :]
        q_ref[...] = acc[:, :DH]
        kv_ref[...] = acc[:, DH:3 * DH]
        s_ref[...] = acc[:, 3 * DH:]

    return pl.pallas_call(
        body,
        grid=(N // RB,),
        in_specs=[
            pl.BlockSpec((RB, DH), lambda i: (i, 0)),
            pl.BlockSpec((DH, 4 * DH), lambda i: (0, 0)),
            pl.BlockSpec((8, 4 * DH), lambda i: (0, 0)),
        ],
        out_specs=[
            pl.BlockSpec((RB, DH), lambda i: (i, 0)),
            pl.BlockSpec((RB, 2 * DH), lambda i: (i, 0)),
            pl.BlockSpec((RB, DH), lambda i: (i, 0)),
        ],
        out_shape=[
            jax.ShapeDtypeStruct((N, DH), jnp.float32),
            jax.ShapeDtypeStruct((N, 2 * DH), jnp.float32),
            jax.ShapeDtypeStruct((N, DH), jnp.float32),
        ],
    )(h, Wcat, bcat8)


def _edge_math(qd, kvg, attrP, WeP, beP):
    ne = qd.shape[0]
    EB = 4000
    inv = 1.0 / (DH ** 0.5)

    def body(qd_ref, kv_ref, at_ref, we_ref, be_ref, msg_ref, ex_ref):
        e = jnp.dot(at_ref[...], we_ref[...], precision=_HIGH) + be_ref[0:1, :]
        kj = kv_ref[:, :DH] + e
        alpha = jnp.sum(qd_ref[...] * kj, axis=1, keepdims=True) * inv
        ex = jnp.exp(alpha)
        msg_ref[...] = (kv_ref[:, DH:] + e) * ex
        ex_ref[...] = ex * jnp.ones((1, DH), jnp.float32)

    return pl.pallas_call(
        body,
        grid=(ne // EB,),
        in_specs=[
            pl.BlockSpec((EB, DH), lambda i: (i, 0)),
            pl.BlockSpec((EB, 2 * DH), lambda i: (i, 0)),
            pl.BlockSpec((EB, 8), lambda i: (i, 0)),
            pl.BlockSpec((8, DH), lambda i: (0, 0)),
            pl.BlockSpec((8, DH), lambda i: (0, 0)),
        ],
        out_specs=[
            pl.BlockSpec((EB, DH), lambda i: (i, 0)),
            pl.BlockSpec((EB, DH), lambda i: (i, 0)),
        ],
        out_shape=[
            jax.ShapeDtypeStruct((ne, DH), jnp.float32),
            jax.ShapeDtypeStruct((ne, DH), jnp.float32),
        ],
    )(qd, kvg, attrP, WeP, beP)


def _combine(omA, omB, oeA, oeB, s):
    def body(a_ref, b_ref, c_ref, d_ref, s_ref, o_ref):
        denom = c_ref[:, 0:1] + d_ref[:, 0:1]
        o_ref[...] = jnp.maximum(
            (a_ref[...] + b_ref[...]) / (denom + 1e-16) + s_ref[...], 0.0)

    return pl.pallas_call(
        body,
        out_shape=jax.ShapeDtypeStruct((N, DH), jnp.float32),
    )(omA, omB, oeA, oeB, s)


def _readout_mlp(H, batch8, batchb, lin1_W, lin1_b8, lin2_W, lin2_b8,
                 lin3_W8, lin3_b8):
    def body(h_ref, b8_ref, bb_ref, w1_ref, b1_ref, w2_ref, b2_ref,
             w3_ref, b3_ref, o_ref):
        H_ = h_ref[...]
        brow = b8_ref[0:1, :]
        gid = lax.broadcasted_iota(jnp.int32, (NG, 1), 0)
        onehot = (brow == gid).astype(jnp.float32)
        counts = jnp.sum(onehot, axis=1, keepdims=True)
        gsum = jnp.dot(onehot, H_, precision=_HIGH)
        gap = gsum / jnp.maximum(counts, 1.0)
        bb = bb_ref[...]
        neg = jnp.float32(-3.0e38)
        rows = []
        for g in range(NG):
            mg = jnp.where(bb == g, H_, neg)
            rows.append(jnp.max(mg, axis=0, keepdims=True))
        gmp = jnp.concatenate(rows, axis=0)
        gmp = jnp.where(gmp > -1.0e38, gmp, 0.0)
        ro = jnp.concatenate([gap, gmp], axis=1)
        o1 = jnp.maximum(jnp.dot(ro, w1_ref[...], precision=_HIGH)
                         + b1_ref[0:1, :], 0.0)
        o2 = jnp.maximum(jnp.dot(o1, w2_ref[...], precision=_HIGH)
                         + b2_ref[0:1, :], 0.0)
        o3 = jnp.dot(o2, w3_ref[...], precision=_HIGH) + b3_ref[0:1, :]
        o_ref[...] = 1.0 / (1.0 + jnp.exp(-o3))

    return pl.pallas_call(
        body,
        out_shape=jax.ShapeDtypeStruct((NG, 8), jnp.float32),
    )(H, batch8, batchb, lin1_W, lin1_b8, lin2_W, lin2_b8, lin3_W8, lin3_b8)


def _conv_layer(h, srcs, dsts, attrs, p, zm):
    Wcat = jnp.concatenate([p["Wq"], p["Wk"], p["Wv"], p["Ws"]], axis=1)
    bcat = jnp.concatenate([p["bq"], p["bk"], p["bv"], p["bs"]])
    bcat8 = jnp.broadcast_to(bcat[None, :], (8, 4 * DH))
    WeP = jnp.zeros((8, DH), jnp.float32).at[:6].set(p["We"])
    beP = jnp.broadcast_to(p["be"][None, :], (8, DH))

    q, kv, s = _qkvs(h, Wcat, bcat8)
    qd1, kvg1 = _sc_gather_edges(q, kv, dsts[0], srcs[0])
    msg1, ex1 = _edge_math(qd1, kvg1, attrs[0], WeP, beP)
    qd2, kvg2 = _sc_gather_edges(q, kv, dsts[1], srcs[1])
    omA, oeA = _sc_scatter_add(msg1, ex1, dsts[0], zm)
    msg2, ex2 = _edge_math(qd2, kvg2, attrs[1], WeP, beP)
    omB, oeB = _sc_scatter_add(msg2, ex2, dsts[1], zm)
    return _combine(omA, omB, oeA, oeB, s)


def kernel(x, edge_index_dir, edge_attr, edge_index_und, edge_attr_und,
           batch, emb, und1, und2, lin1_W, lin1_b, lin2_W, lin2_b,
           lin3_W, lin3_b):
    src = edge_index_und[0].astype(jnp.int32)
    dst = edge_index_und[1].astype(jnp.int32)
    xf = x.reshape(-1).astype(jnp.int32)
    attrP = jnp.concatenate(
        [edge_attr_und, jnp.zeros((E, 2), jnp.float32)], axis=1)
    E2 = E // 2
    srcs = (src[:E2], src[E2:])
    dsts = (dst[:E2], dst[E2:])
    attrs = (attrP[:E2], attrP[E2:])
    zm = jnp.zeros((N, DH), jnp.float32)

    emb128 = jnp.concatenate(
        [emb, jnp.zeros((emb.shape[0], DH - emb.shape[1]), jnp.float32)],
        axis=1)
    h0 = _sc_gather(emb128, xf)[:, :32].reshape(N, DH)
    h1 = _conv_layer(h0, srcs, dsts, attrs, und1, zm)
    h2 = _conv_layer(h1, srcs, dsts, attrs, und2, zm)

    batch8 = jnp.broadcast_to(batch.astype(jnp.int32)[None, :], (8, N))
    batchb = jnp.broadcast_to(batch.astype(jnp.int32)[:, None], (N, DH))
    lin1_b8 = jnp.broadcast_to(lin1_b[None, :], (8, 256))
    lin2_b8 = jnp.broadcast_to(lin2_b[None, :], (8, 128))
    lin3_W8 = jnp.broadcast_to(lin3_W, (128, 8))
    lin3_b8 = jnp.broadcast_to(lin3_b[None, :], (8, 8))
    out = _readout_mlp(h2, batch8, batchb, lin1_W, lin1_b8, lin2_W,
                       lin2_b8, lin3_W8, lin3_b8)
    return out[:, 0]
